# baseline (speedup 1.0000x reference)
import functools

import jax
import jax.numpy as jnp
from jax import lax
from jax.experimental import pallas as pl
from jax.experimental.pallas import tpu as pltpu
from jax.experimental.pallas import tpu_sc as plsc

_BATCH = 4096
_HIST = 50
_DIM = 64
_NW = 32
_CHUNK = 128
_ROWS_PER_W = (_BATCH * _HIST) // _NW
_NCHUNK = _ROWS_PER_W // _CHUNK


def _build_gather():
    mesh = plsc.VectorSubcoreMesh(core_axis_name="c", subcore_axis_name="s")

    @functools.partial(
        pl.kernel,
        out_type=jax.ShapeDtypeStruct((_NW, _NCHUNK, _CHUNK, _DIM), jnp.float32),
        mesh=mesh,
        scratch_types=[
            pltpu.VMEM((_NCHUNK, _CHUNK), jnp.int32),
            pltpu.VMEM((_CHUNK, _DIM), jnp.float32),
            pltpu.SemaphoreType.DMA,
        ],
        compiler_params=pltpu.CompilerParams(use_tc_tiling_on_sc=False),
    )
    def gather_kernel(table_hbm, idx_hbm, out_hbm, idx_v, rows_v, sem):
        wid = lax.axis_index("s") * 2 + lax.axis_index("c")
        pltpu.sync_copy(idx_hbm.at[wid], idx_v)

        def body(j, carry):
            pltpu.async_copy(table_hbm.at[idx_v.at[j]], rows_v, sem).wait()
            pltpu.sync_copy(rows_v, out_hbm.at[wid, j])
            return carry

        lax.fori_loop(0, _NCHUNK, body, 0)

    return gather_kernel


_gather = _build_gather()


def kernel(inputs, embeddings):
    idx = inputs.astype(jnp.int32).reshape(_NW, _NCHUNK, _CHUNK)
    out = _gather(embeddings, idx)
    return out.reshape(_BATCH, _HIST, _DIM)


# speedup vs baseline: 1.0479x; 1.0479x over previous
import functools

import jax
import jax.numpy as jnp
from jax import lax
from jax.experimental import pallas as pl
from jax.experimental.pallas import tpu as pltpu
from jax.experimental.pallas import tpu_sc as plsc

_BATCH = 4096
_HIST = 50
_DIM = 64
_NW = 32
_CHUNK = 128
_ROWS_PER_W = (_BATCH * _HIST) // _NW
_NCHUNK = _ROWS_PER_W // _CHUNK
_NBUF = 8
_DEPTH = 4


def _build_gather():
    mesh = plsc.VectorSubcoreMesh(core_axis_name="c", subcore_axis_name="s")

    @functools.partial(
        pl.kernel,
        out_type=jax.ShapeDtypeStruct((_NW, _NCHUNK, _CHUNK, _DIM), jnp.float32),
        mesh=mesh,
        scratch_types=[
            pltpu.VMEM((_NCHUNK, _CHUNK), jnp.int32),
            pltpu.VMEM((_NBUF, _CHUNK, _DIM), jnp.float32),
            pltpu.SemaphoreType.DMA((_NBUF,)),
            pltpu.SemaphoreType.DMA((_NBUF,)),
        ],
        compiler_params=pltpu.CompilerParams(use_tc_tiling_on_sc=False),
    )
    def gather_kernel(table_hbm, idx_hbm, out_hbm, idx_v, rows_v, gsem, osem):
        wid = lax.axis_index("s") * 2 + lax.axis_index("c")
        pltpu.sync_copy(idx_hbm.at[wid], idx_v)

        def body(j, carry):
            slot_f = j % _NBUF

            @pl.when(j < _NCHUNK)
            def _fire():
                @pl.when(j >= _NBUF)
                def _drain():
                    pltpu.make_async_copy(
                        rows_v.at[slot_f], out_hbm.at[wid, j - _NBUF],
                        osem.at[slot_f]).wait()

                pltpu.make_async_copy(
                    table_hbm.at[idx_v.at[j]], rows_v.at[slot_f],
                    gsem.at[slot_f]).start()

            jj = j - _DEPTH

            @pl.when(jj >= 0)
            def _retire():
                slot_d = jj % _NBUF
                pltpu.make_async_copy(
                    table_hbm.at[idx_v.at[jj]], rows_v.at[slot_d],
                    gsem.at[slot_d]).wait()
                pltpu.make_async_copy(
                    rows_v.at[slot_d], out_hbm.at[wid, jj],
                    osem.at[slot_d]).start()

            return carry

        lax.fori_loop(0, _NCHUNK + _DEPTH, body, 0)

        def tail(j, carry):
            slot = j % _NBUF
            pltpu.make_async_copy(
                rows_v.at[slot], out_hbm.at[wid, j], osem.at[slot]).wait()
            return carry

        lax.fori_loop(_NCHUNK - _NBUF, _NCHUNK, tail, 0)

    return gather_kernel


_gather = _build_gather()


def kernel(inputs, embeddings):
    idx = inputs.astype(jnp.int32).reshape(_NW, _NCHUNK, _CHUNK)
    out = _gather(embeddings, idx)
    return out.reshape(_BATCH, _HIST, _DIM)
